# baseline (speedup 1.0000x reference)
import functools

import jax
import jax.numpy as jnp
from jax.experimental import pallas as pl
from jax.experimental.pallas import tpu as pltpu

_NEG_SLOPE = 0.01
_B = 8
_TW = 4


def _chain_kernel(x_ref, w_ref, b_ref, o_ref, act_a, act_b, *, H, W, C, depth):
    BH = x_ref.shape[0]
    WP = x_ref.shape[1]
    WC = W * C
    LP = C
    S = _TW * C
    tiles = W // _TW

    for buf in (act_a, act_b):
        buf[:, 0:LP] = jnp.zeros((BH, LP), buf.dtype)
        buf[:, LP + WC:] = jnp.zeros((BH, WP - LP - WC), buf.dtype)

    i = jax.lax.broadcasted_iota(jnp.int32, (BH, 1), 0)
    m_up = ((i % H) != 0).astype(jnp.float32)
    m_dn = ((i % H) != (H - 1)).astype(jnp.float32)

    srcs = (x_ref, act_a, act_b, act_a)
    dsts = (act_a, act_b, act_a, None)
    for layer in range(depth):
        src = srcs[layer]
        dst = dsts[layer]
        for t in range(tiles):
            p = jnp.dot(src[:, pl.ds(t * S, 2 * S)], w_ref[layer],
                        preferred_element_type=jnp.float32)
            a0 = m_up * jnp.roll(p[:, 0:S], 1, axis=0)
            a2 = m_dn * jnp.roll(p[:, 2 * S:3 * S], -1, axis=0)
            acc = a0 + p[:, S:2 * S] + a2 + b_ref[layer, 0:1, pl.ds(t * S, S)]
            acc = jnp.maximum(acc, _NEG_SLOPE * acc)
            if layer == depth - 1:
                o_ref[:, pl.ds(t * S, S)] = acc
            else:
                dst[:, pl.ds(LP + t * S, S)] = acc.astype(dst.dtype)


def _fold_w(w, in_c4, out_c4):
    ci, co = w.shape[2], w.shape[3]
    m = jnp.zeros((8, ci, 3, _TW, co), jnp.float32)
    for j in range(_TW):
        for dx in range(3):
            m = m.at[j + dx, :, :, j, :].set(jnp.transpose(w[:, dx], (1, 0, 2)))
    if in_c4:
        m = m.reshape(2, 4, ci, 3, _TW, co).transpose(0, 2, 1, 3, 4, 5)
        m = m.reshape(8 * ci, 3, _TW, co)
    else:
        m = m.reshape(8 * ci, 3, _TW, co)
    if out_c4:
        m = m.transpose(0, 1, 3, 2)
    return m.reshape(8 * ci, 3 * _TW * co)


def kernel(x_nchw, w0, b0, w1, b1, w2, b2, w3, b3):
    params = [(w0, b0), (w1, b1), (w2, b2), (w3, b3)]
    N, C, H, W = x_nchw.shape
    depth = len(params)
    WC = W * C
    WP = WC + 4 * C
    G = W // _TW

    xp = jnp.pad(x_nchw, ((0, 0), (0, 0), (0, 0), (1, 3)))
    xp = xp.reshape(N, C, H, G + 1, _TW)
    xp = jnp.transpose(xp, (0, 2, 3, 1, 4)).astype(jnp.bfloat16)
    x = xp.reshape(N * H, WP)

    ws = jnp.stack(
        [_fold_w(w, in_c4=(l == 0), out_c4=(l == depth - 1))
         for l, (w, _) in enumerate(params)]).astype(jnp.bfloat16)
    bs = jnp.stack(
        [jnp.tile(jnp.repeat(b, _TW), G).reshape(1, WC) if l == depth - 1
         else jnp.tile(b, W).reshape(1, WC)
         for l, (_, b) in enumerate(params)]).astype(jnp.float32)

    B = _B
    BH = B * H
    out = pl.pallas_call(
        functools.partial(_chain_kernel, H=H, W=W, C=C, depth=depth),
        out_shape=jax.ShapeDtypeStruct((N * H, WC), jnp.float32),
        grid=(N // B,),
        in_specs=[
            pl.BlockSpec((BH, WP), lambda n: (n, 0)),
            pl.BlockSpec((depth, 8 * C, 3 * _TW * C), lambda n: (0, 0, 0)),
            pl.BlockSpec((depth, 1, WC), lambda n: (0, 0, 0)),
        ],
        out_specs=pl.BlockSpec((BH, WC), lambda n: (n, 0)),
        scratch_shapes=[
            pltpu.VMEM((BH, WP), jnp.bfloat16),
            pltpu.VMEM((BH, WP), jnp.bfloat16),
        ],
        compiler_params=pltpu.CompilerParams(
            dimension_semantics=("parallel",),
            vmem_limit_bytes=64 * 1024 * 1024),
    )(x, ws, bs)

    out = out.reshape(N, H, G, C, _TW)
    out = jnp.transpose(out, (0, 3, 1, 2, 4)).reshape(N, C, H, W)
    return out


# speedup vs baseline: 1.0308x; 1.0308x over previous
import functools

import jax
import jax.numpy as jnp
from jax.experimental import pallas as pl
from jax.experimental.pallas import tpu as pltpu

_NEG_SLOPE = 0.01
_B = 8
_TW = 4


def _chain_kernel(x_ref, w_ref, b_ref, o_ref, act_a, act_b, *, H, W, C, depth):
    BH = x_ref.shape[0]
    WP = act_a.shape[1]
    WC = W * C
    LP = C
    S = _TW * C
    tiles = W // _TW

    for buf in (act_a, act_b):
        buf[:, 0:LP] = jnp.zeros((BH, LP), buf.dtype)
        buf[:, LP + WC:] = jnp.zeros((BH, WP - LP - WC), buf.dtype)
    act_a[:, LP:LP + WC] = x_ref[...]

    i = jax.lax.broadcasted_iota(jnp.int32, (BH, 1), 0)
    m_up = ((i % H) != 0).astype(jnp.float32)
    m_dn = ((i % H) != (H - 1)).astype(jnp.float32)

    srcs = (act_a, act_b, act_a, act_b)
    dsts = (act_b, act_a, act_b, None)
    for layer in range(depth):
        src = srcs[layer]
        dst = dsts[layer]
        for t in range(tiles):
            p = jnp.dot(src[:, pl.ds(t * S, 2 * S)], w_ref[layer],
                        preferred_element_type=jnp.float32)
            a0 = m_up * jnp.roll(p[:, 0:S], 1, axis=0)
            a2 = m_dn * jnp.roll(p[:, 2 * S:3 * S], -1, axis=0)
            acc = a0 + p[:, S:2 * S] + a2 + b_ref[layer, 0:1, pl.ds(t * S, S)]
            acc = jnp.maximum(acc, _NEG_SLOPE * acc)
            if layer == depth - 1:
                o_ref[:, pl.ds(t * S, S)] = acc
            else:
                dst[:, pl.ds(LP + t * S, S)] = acc.astype(dst.dtype)


def _fold_w(w):
    ci, co = w.shape[2], w.shape[3]
    m = jnp.zeros((8, ci, 3, _TW, co), jnp.float32)
    for j in range(_TW):
        for dx in range(3):
            m = m.at[j + dx, :, :, j, :].set(jnp.transpose(w[:, dx], (1, 0, 2)))
    return m.reshape(8 * ci, 3 * _TW * co)


def kernel(x_nchw, w0, b0, w1, b1, w2, b2, w3, b3):
    params = [(w0, b0), (w1, b1), (w2, b2), (w3, b3)]
    N, C, H, W = x_nchw.shape
    depth = len(params)
    WC = W * C
    WP = WC + 4 * C

    x = jnp.transpose(x_nchw.astype(jnp.bfloat16), (0, 2, 1, 3))
    x = jnp.swapaxes(x.reshape(N * H, C, W), 1, 2).reshape(N * H, WC)

    ws = jnp.stack([_fold_w(w) for w, _ in params]).astype(jnp.bfloat16)
    bs = jnp.stack([jnp.tile(b, W).reshape(1, WC)
                    for _, b in params]).astype(jnp.float32)

    B = _B
    BH = B * H
    out = pl.pallas_call(
        functools.partial(_chain_kernel, H=H, W=W, C=C, depth=depth),
        out_shape=jax.ShapeDtypeStruct((N * H, WC), jnp.float32),
        grid=(N // B,),
        in_specs=[
            pl.BlockSpec((BH, WC), lambda n: (n, 0)),
            pl.BlockSpec((depth, 8 * C, 3 * _TW * C), lambda n: (0, 0, 0)),
            pl.BlockSpec((depth, 1, WC), lambda n: (0, 0, 0)),
        ],
        out_specs=pl.BlockSpec((BH, WC), lambda n: (n, 0)),
        scratch_shapes=[
            pltpu.VMEM((BH, WP), jnp.bfloat16),
            pltpu.VMEM((BH, WP), jnp.bfloat16),
        ],
        compiler_params=pltpu.CompilerParams(
            dimension_semantics=("parallel",),
            vmem_limit_bytes=64 * 1024 * 1024),
    )(x, ws, bs)

    out = jnp.swapaxes(out.reshape(N * H, W, C), 1, 2).reshape(N, H, C, W)
    return jnp.transpose(out, (0, 2, 1, 3))


# speedup vs baseline: 1.1968x; 1.1609x over previous
import functools

import jax
import jax.numpy as jnp
from jax.experimental import pallas as pl
from jax.experimental.pallas import tpu as pltpu

_NEG_SLOPE = 0.01
_B = 16
_TW = 4


def _chain_kernel(x_ref, w_ref, b_ref, o_ref, act_a, act_b, *, H, W, C, depth):
    BH = x_ref.shape[0]
    WP = act_a.shape[1]
    WC = W * C
    LP = C
    S = _TW * C
    tiles = W // _TW

    for buf in (act_a, act_b):
        buf[:, 0:LP] = jnp.zeros((BH, LP), buf.dtype)
        buf[:, LP + WC:] = jnp.zeros((BH, WP - LP - WC), buf.dtype)
    act_a[:, LP:LP + WC] = x_ref[...]

    i = jax.lax.broadcasted_iota(jnp.int32, (BH, 1), 0)
    m_up = ((i % H) != 0).astype(jnp.float32)
    m_dn = ((i % H) != (H - 1)).astype(jnp.float32)

    srcs = (act_a, act_b, act_a, act_b)
    dsts = (act_b, act_a, act_b, None)
    for layer in range(depth):
        src = srcs[layer]
        dst = dsts[layer]
        for t in range(tiles):
            p = jnp.dot(src[:, pl.ds(t * S, 2 * S)], w_ref[layer],
                        preferred_element_type=jnp.float32)
            a0 = m_up * jnp.roll(p[:, 0:S], 1, axis=0)
            a2 = m_dn * jnp.roll(p[:, 2 * S:3 * S], -1, axis=0)
            acc = a0 + p[:, S:2 * S] + a2 + b_ref[layer, 0:1, pl.ds(t * S, S)]
            acc = jnp.maximum(acc, _NEG_SLOPE * acc)
            if layer == depth - 1:
                o_ref[:, pl.ds(t * S, S)] = acc
            else:
                dst[:, pl.ds(LP + t * S, S)] = acc.astype(dst.dtype)


def _fold_w(w):
    ci, co = w.shape[2], w.shape[3]
    m = jnp.zeros((8, ci, 3, _TW, co), jnp.float32)
    for j in range(_TW):
        for dx in range(3):
            m = m.at[j + dx, :, :, j, :].set(jnp.transpose(w[:, dx], (1, 0, 2)))
    return m.reshape(8 * ci, 3 * _TW * co)


def kernel(x_nchw, w0, b0, w1, b1, w2, b2, w3, b3):
    params = [(w0, b0), (w1, b1), (w2, b2), (w3, b3)]
    N, C, H, W = x_nchw.shape
    depth = len(params)
    WC = W * C
    WP = WC + 4 * C

    x = jnp.transpose(x_nchw, (0, 2, 3, 1)).astype(jnp.bfloat16)
    x = x.reshape(N * H, WC)

    ws = jnp.stack([_fold_w(w) for w, _ in params]).astype(jnp.bfloat16)
    bs = jnp.stack([jnp.tile(b, W).reshape(1, WC)
                    for _, b in params]).astype(jnp.float32)

    B = _B
    BH = B * H
    out = pl.pallas_call(
        functools.partial(_chain_kernel, H=H, W=W, C=C, depth=depth),
        out_shape=jax.ShapeDtypeStruct((N * H, WC), jnp.float32),
        grid=(N // B,),
        in_specs=[
            pl.BlockSpec((BH, WC), lambda n: (n, 0)),
            pl.BlockSpec((depth, 8 * C, 3 * _TW * C), lambda n: (0, 0, 0)),
            pl.BlockSpec((depth, 1, WC), lambda n: (0, 0, 0)),
        ],
        out_specs=pl.BlockSpec((BH, WC), lambda n: (n, 0)),
        scratch_shapes=[
            pltpu.VMEM((BH, WP), jnp.bfloat16),
            pltpu.VMEM((BH, WP), jnp.bfloat16),
        ],
        compiler_params=pltpu.CompilerParams(
            dimension_semantics=("parallel",),
            vmem_limit_bytes=64 * 1024 * 1024),
    )(x, ws, bs)

    out = out.reshape(N, H, W, C)
    return jnp.transpose(out, (0, 3, 1, 2))


# speedup vs baseline: 1.3600x; 1.1364x over previous
import functools

import jax
import jax.numpy as jnp
from jax.experimental import pallas as pl
from jax.experimental.pallas import tpu as pltpu

_NEG_SLOPE = 0.01
_B = 8
_TW = 4


def _chain_kernel(x_ref, w_ref, b_ref, o_ref, act_a, act_b, *, H, W, C, depth):
    BH = x_ref.shape[0]
    WP = act_a.shape[1]
    WC = W * C
    LP = C
    S = _TW * C
    tiles = W // _TW

    for buf in (act_a, act_b):
        buf[:, 0:LP] = jnp.zeros((BH, LP), buf.dtype)
        buf[:, LP + WC:] = jnp.zeros((BH, WP - LP - WC), buf.dtype)

    i = jax.lax.broadcasted_iota(jnp.int32, (BH, 1), 0)
    m_up = ((i % H) != 0).astype(jnp.float32)
    m_dn = ((i % H) != (H - 1)).astype(jnp.float32)

    srcs = (x_ref, act_a, act_b, act_a)
    dsts = (act_a, act_b, act_a, None)
    for layer in range(depth):
        src = srcs[layer]
        dst = dsts[layer]
        for t in range(tiles):
            p = jnp.dot(src[:, pl.ds(t * S, 2 * S)], w_ref[layer],
                        preferred_element_type=jnp.float32)
            a0 = m_up * jnp.roll(p[:, 0:S], 1, axis=0)
            a2 = m_dn * jnp.roll(p[:, 2 * S:3 * S], -1, axis=0)
            acc = a0 + p[:, S:2 * S] + a2 + b_ref[layer, 0:1, pl.ds(t * S, S)]
            acc = jnp.maximum(acc, _NEG_SLOPE * acc)
            if layer == depth - 1:
                o_ref[:, pl.ds(t * S, S)] = acc
            else:
                dst[:, pl.ds(LP + t * S, S)] = acc.astype(dst.dtype)


def _fold_w(w):
    ci, co = w.shape[2], w.shape[3]
    m = jnp.zeros((8, ci, 3, _TW, co), jnp.float32)
    for j in range(_TW):
        for dx in range(3):
            m = m.at[j + dx, :, :, j, :].set(jnp.transpose(w[:, dx], (1, 0, 2)))
    return m.reshape(8 * ci, 3 * _TW * co)


def kernel(x_nchw, w0, b0, w1, b1, w2, b2, w3, b3):
    params = [(w0, b0), (w1, b1), (w2, b2), (w3, b3)]
    N, C, H, W = x_nchw.shape
    depth = len(params)
    WC = W * C
    WP = WC + 4 * C

    x = jnp.transpose(x_nchw, (0, 2, 3, 1)).astype(jnp.bfloat16)
    x = x.reshape(N, H, WC)
    x = jnp.pad(x, ((0, 0), (0, 0), (C, 3 * C))).reshape(N * H, WP)

    ws = jnp.stack([_fold_w(w) for w, _ in params]).astype(jnp.bfloat16)
    bs = jnp.stack([jnp.tile(b, W).reshape(1, WC)
                    for _, b in params]).astype(jnp.float32)

    B = _B
    BH = B * H
    out = pl.pallas_call(
        functools.partial(_chain_kernel, H=H, W=W, C=C, depth=depth),
        out_shape=jax.ShapeDtypeStruct((N * H, WC), jnp.float32),
        grid=(N // B,),
        in_specs=[
            pl.BlockSpec((BH, WP), lambda n: (n, 0)),
            pl.BlockSpec((depth, 8 * C, 3 * _TW * C), lambda n: (0, 0, 0)),
            pl.BlockSpec((depth, 1, WC), lambda n: (0, 0, 0)),
        ],
        out_specs=pl.BlockSpec((BH, WC), lambda n: (n, 0)),
        scratch_shapes=[
            pltpu.VMEM((BH, WP), jnp.bfloat16),
            pltpu.VMEM((BH, WP), jnp.bfloat16),
        ],
        compiler_params=pltpu.CompilerParams(
            dimension_semantics=("parallel",),
            vmem_limit_bytes=64 * 1024 * 1024),
    )(x, ws, bs)

    out = out.reshape(N, H, W, C)
    return jnp.transpose(out, (0, 3, 1, 2))


# speedup vs baseline: 1.3658x; 1.0042x over previous
import functools

import jax
import jax.numpy as jnp
from jax.experimental import pallas as pl
from jax.experimental.pallas import tpu as pltpu

_NEG_SLOPE = 0.01
_B = 16
_TW = 4


def _chain_kernel(x_ref, w_ref, b_ref, o_ref, act_a, act_b, *, H, W, C, depth):
    BH = x_ref.shape[0]
    WP = act_a.shape[1]
    WC = W * C
    LP = C
    S = _TW * C
    tiles = W // _TW

    for buf in (act_a, act_b):
        buf[:, 0:LP] = jnp.zeros((BH, LP), buf.dtype)
        buf[:, LP + WC:] = jnp.zeros((BH, WP - LP - WC), buf.dtype)

    i = jax.lax.broadcasted_iota(jnp.int32, (BH, 1), 0)
    m_up = ((i % H) != 0).astype(jnp.float32)
    m_dn = ((i % H) != (H - 1)).astype(jnp.float32)

    srcs = (x_ref, act_a, act_b, act_a)
    dsts = (act_a, act_b, act_a, None)
    for layer in range(depth):
        src = srcs[layer]
        dst = dsts[layer]
        for t in range(tiles):
            p = jnp.dot(src[:, pl.ds(t * S, 2 * S)], w_ref[layer],
                        preferred_element_type=jnp.float32)
            a0 = m_up * jnp.roll(p[:, 0:S], 1, axis=0)
            a2 = m_dn * jnp.roll(p[:, 2 * S:3 * S], -1, axis=0)
            acc = a0 + p[:, S:2 * S] + a2 + b_ref[layer, 0:1, pl.ds(t * S, S)]
            acc = jnp.maximum(acc, _NEG_SLOPE * acc)
            if layer == depth - 1:
                o_ref[:, pl.ds(t * S, S)] = acc
            else:
                dst[:, pl.ds(LP + t * S, S)] = acc.astype(dst.dtype)


def _fold_w(w):
    ci, co = w.shape[2], w.shape[3]
    m = jnp.zeros((8, ci, 3, _TW, co), jnp.float32)
    for j in range(_TW):
        for dx in range(3):
            m = m.at[j + dx, :, :, j, :].set(jnp.transpose(w[:, dx], (1, 0, 2)))
    return m.reshape(8 * ci, 3 * _TW * co)


def kernel(x_nchw, w0, b0, w1, b1, w2, b2, w3, b3):
    params = [(w0, b0), (w1, b1), (w2, b2), (w3, b3)]
    N, C, H, W = x_nchw.shape
    depth = len(params)
    WC = W * C
    WP = WC + 4 * C

    x = jnp.transpose(x_nchw, (0, 2, 3, 1)).astype(jnp.bfloat16)
    x = x.reshape(N, H, WC)
    x = jnp.pad(x, ((0, 0), (0, 0), (C, 3 * C))).reshape(N * H, WP)

    ws = jnp.stack([_fold_w(w) for w, _ in params]).astype(jnp.bfloat16)
    bs = jnp.stack([jnp.tile(b, W).reshape(1, WC)
                    for _, b in params]).astype(jnp.float32)

    B = _B
    BH = B * H
    out = pl.pallas_call(
        functools.partial(_chain_kernel, H=H, W=W, C=C, depth=depth),
        out_shape=jax.ShapeDtypeStruct((N * H, WC), jnp.float32),
        grid=(N // B,),
        in_specs=[
            pl.BlockSpec((BH, WP), lambda n: (n, 0)),
            pl.BlockSpec((depth, 8 * C, 3 * _TW * C), lambda n: (0, 0, 0)),
            pl.BlockSpec((depth, 1, WC), lambda n: (0, 0, 0)),
        ],
        out_specs=pl.BlockSpec((BH, WC), lambda n: (n, 0)),
        scratch_shapes=[
            pltpu.VMEM((BH, WP), jnp.bfloat16),
            pltpu.VMEM((BH, WP), jnp.bfloat16),
        ],
        compiler_params=pltpu.CompilerParams(
            dimension_semantics=("parallel",),
            vmem_limit_bytes=64 * 1024 * 1024),
    )(x, ws, bs)

    out = out.reshape(N, H, W, C)
    return jnp.transpose(out, (0, 3, 1, 2))
